# Initial kernel scaffold; baseline (speedup 1.0000x reference)
import jax
import jax.numpy as jnp
from jax.experimental import pallas as pl


def kernel(X, D, enc_W, latent_bias, pre_bias):
    raise NotImplementedError("write your pallas kernel here")


# speedup vs baseline: 1.7551x; 1.7551x over previous
import jax
import jax.numpy as jnp
from jax.experimental import pallas as pl

TOPK = 64
_INT_MIN = -2147483648


def _encode_body(x_ref, pb_ref, w_ref, lb_ref, out_ref):
    xc = x_ref[...] - pb_ref[...]
    w = w_ref[...]
    acc = jax.lax.dot_general(xc, w, (((1,), (1,)), ((), ())),
                              preferred_element_type=jnp.float32)
    out_ref[...] = acc + lb_ref[...]


def _topk_body(sp_ref, s_ref):
    v = sp_ref[...]
    i = jax.lax.bitcast_convert_type(v, jnp.int32)
    int_min = jnp.int32(_INT_MIN)
    s = jnp.where(i < 0, int_min - i, i)
    rb = v.shape[0]

    def body(t, u_lo):
        bit = jax.lax.shift_left(jnp.int32(1), jnp.int32(31) - t)
        cand_u = jax.lax.bitwise_or(u_lo, bit)
        cand = int_min + cand_u
        cnt = jnp.sum((s >= cand).astype(jnp.int32), axis=1, keepdims=True)
        return jnp.where(cnt >= TOPK, cand_u, u_lo)

    u_lo = jax.lax.fori_loop(0, 32, body, jnp.zeros((rb, 1), jnp.int32))
    thr = int_min + u_lo
    mask = s >= thr
    s_ref[...] = jnp.where(mask & (v > 0), v, 0.0)


def _decode_body(s_ref, d_ref, pb_ref, out_ref):
    step = pl.program_id(0)

    @pl.when(step == 0)
    def _():
        out_ref[...] = jnp.broadcast_to(pb_ref[...], out_ref.shape)

    d = d_ref[...]
    nrm2 = jnp.sum(d * d, axis=1, keepdims=True)
    dn = d * jax.lax.rsqrt(nrm2)
    out_ref[...] += jax.lax.dot_general(s_ref[...], dn, (((1,), (0,)), ((), ())),
                                        preferred_element_type=jnp.float32)


def kernel(X, D, enc_W, latent_bias, pre_bias):
    T, M = X.shape
    N = enc_W.shape[0]
    lb2 = latent_bias.reshape(1, N)
    pb2 = pre_bias.reshape(1, M)

    TN = 256
    S_pre = pl.pallas_call(
        _encode_body,
        grid=(N // TN,),
        in_specs=[
            pl.BlockSpec((T, M), lambda i: (0, 0)),
            pl.BlockSpec((1, M), lambda i: (0, 0)),
            pl.BlockSpec((TN, M), lambda i: (i, 0)),
            pl.BlockSpec((1, TN), lambda i: (0, i)),
        ],
        out_specs=pl.BlockSpec((T, TN), lambda i: (0, i)),
        out_shape=jax.ShapeDtypeStruct((T, N), jnp.float32),
    )(X, pb2, enc_W, lb2)

    RB = 8
    S_ = pl.pallas_call(
        _topk_body,
        grid=(T // RB,),
        in_specs=[pl.BlockSpec((RB, N), lambda i: (i, 0))],
        out_specs=pl.BlockSpec((RB, N), lambda i: (i, 0)),
        out_shape=jax.ShapeDtypeStruct((T, N), jnp.float32),
    )(S_pre)

    TD = 256
    X_ = pl.pallas_call(
        _decode_body,
        grid=(N // TD,),
        in_specs=[
            pl.BlockSpec((T, TD), lambda i: (0, i)),
            pl.BlockSpec((TD, M), lambda i: (i, 0)),
            pl.BlockSpec((1, M), lambda i: (0, 0)),
        ],
        out_specs=pl.BlockSpec((T, M), lambda i: (0, 0)),
        out_shape=jax.ShapeDtypeStruct((T, M), jnp.float32),
    )(S_, D, pb2)

    return (S_, X_)
